# baseline (speedup 1.0000x reference)
import functools

import jax
import jax.numpy as jnp
from jax.experimental import pallas as pl


M_BLK = 512
K_TILE = 2048
K = 8192
D = 64


def _vq_kernel(x_ref, emb_ref, idx_ref, q_ref, ssq_ref):
    x = x_ref[...]
    x_sq = jnp.sum(x * x, axis=1, keepdims=True)

    n_tiles = K // K_TILE

    def dist_body(k, carry):
        run_min, run_idx = carry
        et = emb_ref[pl.ds(k * K_TILE, K_TILE), :]
        esq = jnp.sum(et * et, axis=1)
        xe2 = 2.0 * jax.lax.dot_general(
            x.astype(jnp.bfloat16), et.astype(jnp.bfloat16),
            (((1,), (1,)), ((), ())),
            preferred_element_type=jnp.float32)
        d = (x_sq - xe2) + esq[None, :]
        tmin = jnp.min(d, axis=1, keepdims=True)
        ids = jax.lax.broadcasted_iota(jnp.int32, (M_BLK, K_TILE), 1) + k * K_TILE
        targ = jnp.min(jnp.where(d <= tmin, ids, K), axis=1, keepdims=True)
        upd = tmin < run_min
        new_min = jnp.where(upd, tmin, run_min)
        new_min = new_min.astype(jnp.bfloat16).astype(jnp.float32)
        return (new_min, jnp.where(upd, targ, run_idx))

    run_min = jnp.full((M_BLK, 1), jnp.inf, jnp.float32)
    run_idx = jnp.zeros((M_BLK, 1), jnp.int32)
    run_min, run_idx = jax.lax.fori_loop(0, n_tiles, dist_body,
                                         (run_min, run_idx))
    idx_ref[...] = run_idx

    def gather_body(k, acc):
        et = emb_ref[pl.ds(k * K_TILE, K_TILE), :]
        ids = jax.lax.broadcasted_iota(jnp.int32, (M_BLK, K_TILE), 1) + k * K_TILE
        oh = (ids == run_idx).astype(jnp.float32)
        return acc + jax.lax.dot_general(
            oh, et, (((1,), (0,)), ((), ())),
            preferred_element_type=jnp.float32)
    q = jax.lax.fori_loop(0, n_tiles, gather_body,
                          jnp.zeros((M_BLK, D), jnp.float32))
    q_ref[...] = q

    diff = q - x
    ssq = jnp.sum(diff * diff, keepdims=True).reshape(1, 1)

    @pl.when(pl.program_id(0) == 0)
    def _():
        ssq_ref[...] = jnp.zeros((1, 1), jnp.float32)
    ssq_ref[...] += ssq


def _vq(x_flat, emb):
    m = x_flat.shape[0]
    grid = (m // M_BLK,)
    idx, q, ssq = pl.pallas_call(
        _vq_kernel,
        grid=grid,
        in_specs=[
            pl.BlockSpec((M_BLK, D), lambda i: (i, 0)),
            pl.BlockSpec((K, D), lambda i: (0, 0)),
        ],
        out_specs=[
            pl.BlockSpec((M_BLK, 1), lambda i: (i, 0)),
            pl.BlockSpec((M_BLK, D), lambda i: (i, 0)),
            pl.BlockSpec((1, 1), lambda i: (0, 0)),
        ],
        out_shape=[
            jax.ShapeDtypeStruct((m, 1), jnp.int32),
            jax.ShapeDtypeStruct((m, D), jnp.float32),
            jax.ShapeDtypeStruct((1, 1), jnp.float32),
        ],
    )(x_flat, emb)
    return idx[:, 0], q, ssq[0, 0]


def _conv(x, w, b, stride, pad):
    out = jax.lax.conv_general_dilated(
        x, w, (stride, stride), ((pad, pad), (pad, pad)),
        dimension_numbers=('NCHW', 'OIHW', 'NCHW'))
    return out + b[None, :, None, None]


def _conv_transpose(x, w, b, stride, pad):
    k = w.shape[2]
    w_t = jnp.flip(w, axis=(2, 3)).transpose(1, 0, 2, 3)
    p = k - 1 - pad
    out = jax.lax.conv_general_dilated(
        x, w_t, (1, 1), ((p, p), (p, p)),
        lhs_dilation=(stride, stride),
        dimension_numbers=('NCHW', 'OIHW', 'NCHW'))
    return out + b[None, :, None, None]


@jax.jit
def kernel(x, conv1_w, conv1_b, conv2_w, conv2_b, emb,
           deconv1_w, deconv1_b, deconv2_w, deconv2_b):
    h1 = jax.nn.relu(_conv(x, conv1_w, conv1_b, 2, 1))
    z_e = _conv(h1, conv2_w, conv2_b, 2, 1)
    b, c, h, w = z_e.shape
    x_flat = z_e.transpose(0, 2, 3, 1).reshape(-1, c)

    idx, q, ssq = _vq(x_flat, emb)

    mse = ssq / jnp.float32(x_flat.size)
    vq_loss = mse * 1.25
    quantized = q.reshape(b, h, w, c).transpose(0, 3, 1, 2)

    d1 = jax.nn.relu(_conv_transpose(quantized, deconv1_w, deconv1_b, 2, 1))
    x_hat = jax.nn.sigmoid(_conv_transpose(d1, deconv2_w, deconv2_b, 2, 1))
    return (x_hat, vq_loss, idx.reshape(b, h, w))
